# Initial kernel scaffold; baseline (speedup 1.0000x reference)
import jax
import jax.numpy as jnp
from jax.experimental import pallas as pl


def kernel(x, pos_edge_index, neg_edge_index, W, b):
    raise NotImplementedError("write your pallas kernel here")


# speedup vs baseline: 5.2801x; 5.2801x over previous
import functools

import jax
import jax.numpy as jnp
from jax import lax
from jax.experimental import pallas as pl
from jax.experimental.pallas import tpu as pltpu
from jax.experimental.pallas import tpu_sc as plsc

N_NODES = 10000
N_EDGES = 320000
D = 128

NC = 2
NS = 16
NW = NC * NS

CHA = 200
CHS = 400
EPT_AGG = N_EDGES // NW
EPT_SCO = (2 * N_EDGES) // NW
ZROWS = 1000
NZT = N_NODES // ZROWS

_mesh = plsc.VectorSubcoreMesh(
    core_axis_name="c", subcore_axis_name="s", num_cores=NC, num_subcores=NS
)


def _mm_body(x_ref, w_ref, b_ref, o_ref):
    o_ref[...] = (
        jnp.dot(x_ref[...], w_ref[...], preferred_element_type=jnp.float32)
        + b_ref[...]
    )


def _matmul(x, W, b):
    return pl.pallas_call(
        _mm_body,
        out_shape=jax.ShapeDtypeStruct((N_NODES, D), jnp.float32),
        grid=(10,),
        in_specs=[
            pl.BlockSpec((N_NODES // 10, D), lambda i: (i, 0)),
            pl.BlockSpec((D, D), lambda i: (0, 0)),
            pl.BlockSpec((1, D), lambda i: (0, 0)),
        ],
        out_specs=pl.BlockSpec((N_NODES // 10, D), lambda i: (i, 0)),
    )(x, W, b.reshape(1, D))


def _agg_body(
    wh_hbm, src_hbm, dst_hbm, zrow_hbm, zdeg_hbm, ones_hbm,
    sum_out, deg_out,
    acc_sh, deg_sh,
    idx_s, idx_d, rows, ones_v, deg_v, sem,
):
    c = lax.axis_index("c")
    s = lax.axis_index("s")

    @pl.when(s < NZT)
    def _():
        pltpu.sync_copy(
            zrow_hbm.at[pl.ds(s * ZROWS, ZROWS)],
            acc_sh.at[pl.ds(s * ZROWS, ZROWS)],
        )

    @pl.when(s == NZT)
    def _():
        pltpu.sync_copy(zdeg_hbm, deg_v)
        pltpu.sync_copy(deg_v, deg_sh)

    pltpu.sync_copy(ones_hbm, ones_v)
    plsc.subcore_barrier()

    wid = s * NC + c
    base_e = wid * EPT_AGG

    def chunk(i, carry):
        off = base_e + i * CHA
        pltpu.sync_copy(src_hbm.at[pl.ds(off, CHA)], idx_s)
        pltpu.sync_copy(dst_hbm.at[pl.ds(off, CHA)], idx_d)
        pltpu.async_copy(wh_hbm.at[idx_s], rows, sem).wait()
        pltpu.sync_copy(rows, acc_sh.at[idx_d], add=True)
        pltpu.sync_copy(ones_v, deg_sh.at[idx_d], add=True)
        return carry

    lax.fori_loop(0, EPT_AGG // CHA, chunk, 0)
    plsc.subcore_barrier()

    @pl.when(s < NZT)
    def _():
        pltpu.sync_copy(
            acc_sh.at[pl.ds(s * ZROWS, ZROWS)],
            sum_out.at[c, pl.ds(s * ZROWS, ZROWS)],
        )

    @pl.when(s == NZT)
    def _():
        pltpu.sync_copy(deg_sh, deg_v)
        pltpu.sync_copy(deg_v, deg_out.at[pl.ds(c * N_NODES, N_NODES)])


_agg = functools.partial(
    pl.kernel,
    out_type=(
        jax.ShapeDtypeStruct((NC, N_NODES, D), jnp.float32),
        jax.ShapeDtypeStruct((NC * N_NODES,), jnp.float32),
    ),
    mesh=_mesh,
    scratch_types=[
        pltpu.VMEM_SHARED((N_NODES, D), jnp.float32),
        pltpu.VMEM_SHARED((N_NODES,), jnp.float32),
        pltpu.VMEM((CHA,), jnp.int32),
        pltpu.VMEM((CHA,), jnp.int32),
        pltpu.VMEM((CHA, D), jnp.float32),
        pltpu.VMEM((CHA,), jnp.float32),
        pltpu.VMEM((N_NODES,), jnp.float32),
        pltpu.SemaphoreType.DMA,
    ],
)(_agg_body)


def _comb_body(s_ref, d_ref, o_ref):
    deg = d_ref[0] + d_ref[1]
    o_ref[...] = (s_ref[0] + s_ref[1]) / jnp.maximum(deg, 1.0)


def _combine(sum_p, deg_p):
    blk = N_NODES // 10
    return pl.pallas_call(
        _comb_body,
        out_shape=jax.ShapeDtypeStruct((N_NODES, D), jnp.float32),
        grid=(10,),
        in_specs=[
            pl.BlockSpec((NC, blk, D), lambda i: (0, i, 0)),
            pl.BlockSpec((NC, blk, 1), lambda i: (0, i, 0)),
        ],
        out_specs=pl.BlockSpec((blk, D), lambda i: (i, 0)),
    )(sum_p, deg_p)


def _score_body(
    h_hbm, src_hbm, dst_hbm,
    out_hbm,
    idx_s, idx_d, ra, rb, accs, sc_v, sem_a, sem_b,
):
    c = lax.axis_index("c")
    s = lax.axis_index("s")
    wid = s * NC + c
    base_e = wid * EPT_SCO
    col = lax.broadcasted_iota(jnp.int32, (16,), 0) * 16

    def chunk(i, carry):
        off = base_e + i * CHS
        pltpu.sync_copy(src_hbm.at[pl.ds(off, CHS)], idx_s)
        pltpu.sync_copy(dst_hbm.at[pl.ds(off, CHS)], idx_d)
        cp_a = pltpu.async_copy(h_hbm.at[idx_s], ra, sem_a)
        cp_b = pltpu.async_copy(h_hbm.at[idx_d], rb, sem_b)
        cp_a.wait()
        cp_b.wait()

        def grp(g, carry2):
            def edge(e, carry3):
                r = g * 16 + e
                acc = ra[r, pl.ds(0, 16)] * rb[r, pl.ds(0, 16)]
                for k in range(1, 8):
                    acc = acc + ra[r, pl.ds(k * 16, 16)] * rb[r, pl.ds(k * 16, 16)]
                accs[pl.ds(e * 16, 16)] = acc
                return carry3

            lax.fori_loop(0, 16, edge, 0)
            tot = plsc.load_gather(accs, [col])
            for j in range(1, 16):
                tot = tot + plsc.load_gather(accs, [col + j])
            sc_v[pl.ds(g * 16, 16)] = tot
            return carry2

        lax.fori_loop(0, CHS // 16, grp, 0)
        pltpu.sync_copy(sc_v, out_hbm.at[pl.ds(off, CHS)])
        return carry

    lax.fori_loop(0, EPT_SCO // CHS, chunk, 0)


_score = functools.partial(
    pl.kernel,
    out_type=jax.ShapeDtypeStruct((2 * N_EDGES,), jnp.float32),
    mesh=_mesh,
    scratch_types=[
        pltpu.VMEM((CHS,), jnp.int32),
        pltpu.VMEM((CHS,), jnp.int32),
        pltpu.VMEM((CHS, D), jnp.float32),
        pltpu.VMEM((CHS, D), jnp.float32),
        pltpu.VMEM((16 * 16,), jnp.float32),
        pltpu.VMEM((CHS,), jnp.float32),
        pltpu.SemaphoreType.DMA,
        pltpu.SemaphoreType.DMA,
    ],
    compiler_params=pltpu.CompilerParams(needs_layout_passes=False),
)(_score_body)


def kernel(x, pos_edge_index, neg_edge_index, W, b):
    Wh = _matmul(x, W, b)

    zrow = jnp.zeros((N_NODES, D), jnp.float32)
    zdeg = jnp.zeros((N_NODES,), jnp.float32)
    ones = jnp.ones((CHA,), jnp.float32)

    sum_p, deg_p = _agg(
        Wh, pos_edge_index[0], pos_edge_index[1], zrow, zdeg, ones
    )
    h = _combine(sum_p, deg_p.reshape(NC, N_NODES, 1))

    all_src = jnp.concatenate([pos_edge_index[0], neg_edge_index[0]])
    all_dst = jnp.concatenate([pos_edge_index[1], neg_edge_index[1]])
    scores = _score(h, all_src, all_dst)

    pos_score = scores[:N_EDGES, None]
    neg_score = scores[N_EDGES:, None]
    return (pos_score, neg_score)


# speedup vs baseline: 7.7116x; 1.4605x over previous
import functools

import jax
import jax.numpy as jnp
from jax import lax
from jax.experimental import pallas as pl
from jax.experimental.pallas import tpu as pltpu
from jax.experimental.pallas import tpu_sc as plsc

N_NODES = 10000
N_EDGES = 320000
D = 128

NC = 2
NS = 16
NW = NC * NS

EPT_AGG = N_EDGES // NW
EPT_SCO = (2 * N_EDGES) // NW
CHA = 80
NCH_A = EPT_AGG // CHA
CHS = 80
NCH_S = EPT_SCO // CHS
ZROWS = 1000
NZT = N_NODES // ZROWS

_mesh = plsc.VectorSubcoreMesh(
    core_axis_name="c", subcore_axis_name="s", num_cores=NC, num_subcores=NS
)


def _mm_body(x_ref, w_ref, b_ref, o_ref):
    o_ref[...] = (
        jnp.dot(x_ref[...], w_ref[...], preferred_element_type=jnp.float32)
        + b_ref[...]
    )


def _matmul(x, W, b):
    return pl.pallas_call(
        _mm_body,
        out_shape=jax.ShapeDtypeStruct((N_NODES, D), jnp.float32),
        grid=(10,),
        in_specs=[
            pl.BlockSpec((N_NODES // 10, D), lambda i: (i, 0)),
            pl.BlockSpec((D, D), lambda i: (0, 0)),
            pl.BlockSpec((1, D), lambda i: (0, 0)),
        ],
        out_specs=pl.BlockSpec((N_NODES // 10, D), lambda i: (i, 0)),
    )(x, W, b.reshape(1, D))


def _agg_body(
    wh_hbm, idx_hbm, zrow_hbm, zdeg_hbm, ones_hbm,
    sum_out, deg_out,
    acc_sh, deg_sh,
    idx0, idx1, rows0, rows1, ones_v, deg_v,
    sI0, sI1, sG0, sG1, sS0, sS1, sD0, sD1,
):
    c = lax.axis_index("c")
    s = lax.axis_index("s")
    wid = s * NC + c

    @pl.when(s < NZT)
    def _():
        pltpu.sync_copy(
            zrow_hbm.at[pl.ds(s * ZROWS, ZROWS)],
            acc_sh.at[pl.ds(s * ZROWS, ZROWS)],
        )

    @pl.when(s == NZT)
    def _():
        pltpu.sync_copy(zdeg_hbm, deg_v)
        pltpu.sync_copy(deg_v, deg_sh)

    pltpu.sync_copy(ones_hbm, ones_v)
    plsc.subcore_barrier()

    def icopy(j, idxb, si):
        pltpu.async_copy(idx_hbm.at[wid, j], idxb, si)

    def iwait(j, idxb, si):
        pltpu.make_async_copy(idx_hbm.at[wid, j], idxb, si).wait()

    def gstart(idxb, rows, sg):
        pltpu.async_copy(wh_hbm.at[idxb.at[0]], rows, sg)

    def gwait(idxb, rows, sg):
        pltpu.make_async_copy(wh_hbm.at[idxb.at[0]], rows, sg).wait()

    def sstart(idxb, rows, ss, sd):
        pltpu.async_copy(rows, acc_sh.at[idxb.at[1]], ss, add=True)
        pltpu.async_copy(ones_v, deg_sh.at[idxb.at[1]], sd, add=True)

    def swait(idxb, rows, ss, sd):
        pltpu.make_async_copy(rows, acc_sh.at[idxb.at[1]], ss).wait()
        pltpu.make_async_copy(ones_v, deg_sh.at[idxb.at[1]], sd).wait()

    icopy(0, idx0, sI0)
    iwait(0, idx0, sI0)
    gstart(idx0, rows0, sG0)

    def body(p, carry):
        i0 = 2 * p
        i1 = i0 + 1
        i2 = i0 + 2

        @pl.when(p > 0)
        def _():
            swait(idx1, rows1, sS1, sD1)

        icopy(i1, idx1, sI1)
        gwait(idx0, rows0, sG0)
        iwait(i1, idx1, sI1)
        gstart(idx1, rows1, sG1)
        sstart(idx0, rows0, sS0, sD0)
        gwait(idx1, rows1, sG1)
        swait(idx0, rows0, sS0, sD0)

        @pl.when(i2 < NCH_A)
        def _():
            icopy(i2, idx0, sI0)
            iwait(i2, idx0, sI0)
            gstart(idx0, rows0, sG0)

        sstart(idx1, rows1, sS1, sD1)
        return carry

    lax.fori_loop(0, NCH_A // 2, body, 0)
    swait(idx1, rows1, sS1, sD1)
    gwait(idx0, rows0, sG0)
    sstart(idx0, rows0, sS0, sD0)
    swait(idx0, rows0, sS0, sD0)
    plsc.subcore_barrier()

    @pl.when(s < NZT)
    def _():
        pltpu.sync_copy(
            acc_sh.at[pl.ds(s * ZROWS, ZROWS)],
            sum_out.at[c, pl.ds(s * ZROWS, ZROWS)],
        )

    @pl.when(s == NZT)
    def _():
        pltpu.sync_copy(deg_sh, deg_v)
        pltpu.sync_copy(deg_v, deg_out.at[pl.ds(c * N_NODES, N_NODES)])


_agg = functools.partial(
    pl.kernel,
    out_type=(
        jax.ShapeDtypeStruct((NC, N_NODES, D), jnp.float32),
        jax.ShapeDtypeStruct((NC * N_NODES,), jnp.float32),
    ),
    mesh=_mesh,
    scratch_types=[
        pltpu.VMEM_SHARED((N_NODES, D), jnp.float32),
        pltpu.VMEM_SHARED((N_NODES,), jnp.float32),
        pltpu.VMEM((2, CHA), jnp.int32),
        pltpu.VMEM((2, CHA), jnp.int32),
        pltpu.VMEM((CHA, D), jnp.float32),
        pltpu.VMEM((CHA, D), jnp.float32),
        pltpu.VMEM((CHA,), jnp.float32),
        pltpu.VMEM((N_NODES,), jnp.float32),
        pltpu.SemaphoreType.DMA,
        pltpu.SemaphoreType.DMA,
        pltpu.SemaphoreType.DMA,
        pltpu.SemaphoreType.DMA,
        pltpu.SemaphoreType.DMA,
        pltpu.SemaphoreType.DMA,
        pltpu.SemaphoreType.DMA,
        pltpu.SemaphoreType.DMA,
    ],
)(_agg_body)


def _comb_body(s_ref, d_ref, o_ref):
    deg = d_ref[0] + d_ref[1]
    o_ref[...] = (s_ref[0] + s_ref[1]) / jnp.maximum(deg, 1.0)


def _combine(sum_p, deg_p):
    blk = N_NODES // 10
    return pl.pallas_call(
        _comb_body,
        out_shape=jax.ShapeDtypeStruct((N_NODES, D), jnp.float32),
        grid=(10,),
        in_specs=[
            pl.BlockSpec((NC, blk, D), lambda i: (0, i, 0)),
            pl.BlockSpec((NC, blk, 1), lambda i: (0, i, 0)),
        ],
        out_specs=pl.BlockSpec((blk, D), lambda i: (i, 0)),
    )(sum_p, deg_p)


def _score_body(
    h_hbm, src_hbm, dst_hbm,
    out_hbm,
    idx_s, idx_d, ra0, rb0, ra1, rb1, accs, sc_buf,
    sA0, sB0, sA1, sB1,
):
    c = lax.axis_index("c")
    s = lax.axis_index("s")
    wid = s * NC + c
    base = wid * EPT_SCO
    pltpu.sync_copy(src_hbm.at[pl.ds(base, EPT_SCO)], idx_s)
    pltpu.sync_copy(dst_hbm.at[pl.ds(base, EPT_SCO)], idx_d)
    col = lax.broadcasted_iota(jnp.int32, (16,), 0) * 16

    def start(j, ra, rb, sa, sb):
        pltpu.async_copy(h_hbm.at[idx_s.at[pl.ds(j * CHS, CHS)]], ra, sa)
        pltpu.async_copy(h_hbm.at[idx_d.at[pl.ds(j * CHS, CHS)]], rb, sb)

    def wait(j, ra, rb, sa, sb):
        pltpu.make_async_copy(
            h_hbm.at[idx_s.at[pl.ds(j * CHS, CHS)]], ra, sa
        ).wait()
        pltpu.make_async_copy(
            h_hbm.at[idx_d.at[pl.ds(j * CHS, CHS)]], rb, sb
        ).wait()

    def compute(j, ra, rb):
        def grp(g, carry):
            for e in range(16):
                r = g * 16 + e
                acc = ra[r, pl.ds(0, 16)] * rb[r, pl.ds(0, 16)]
                for k in range(1, 8):
                    acc = acc + ra[r, pl.ds(k * 16, 16)] * rb[r, pl.ds(k * 16, 16)]
                accs[pl.ds(e * 16, 16)] = acc
            tot = plsc.load_gather(accs, [col])
            for jj in range(1, 16):
                tot = tot + plsc.load_gather(accs, [col + jj])
            sc_buf[pl.ds(j * CHS + g * 16, 16)] = tot
            return carry

        lax.fori_loop(0, CHS // 16, grp, 0)

    start(0, ra0, rb0, sA0, sB0)

    def body(p, carry):
        i0 = 2 * p
        i1 = i0 + 1
        i2 = i0 + 2
        start(i1, ra1, rb1, sA1, sB1)
        wait(i0, ra0, rb0, sA0, sB0)
        compute(i0, ra0, rb0)

        @pl.when(i2 < NCH_S)
        def _():
            start(i2, ra0, rb0, sA0, sB0)

        wait(i1, ra1, rb1, sA1, sB1)
        compute(i1, ra1, rb1)
        return carry

    lax.fori_loop(0, NCH_S // 2, body, 0)
    pltpu.sync_copy(sc_buf, out_hbm.at[pl.ds(base, EPT_SCO)])


_score = functools.partial(
    pl.kernel,
    out_type=jax.ShapeDtypeStruct((2 * N_EDGES,), jnp.float32),
    mesh=_mesh,
    scratch_types=[
        pltpu.VMEM((EPT_SCO,), jnp.int32),
        pltpu.VMEM((EPT_SCO,), jnp.int32),
        pltpu.VMEM((CHS, D), jnp.float32),
        pltpu.VMEM((CHS, D), jnp.float32),
        pltpu.VMEM((CHS, D), jnp.float32),
        pltpu.VMEM((CHS, D), jnp.float32),
        pltpu.VMEM((16 * 16,), jnp.float32),
        pltpu.VMEM((EPT_SCO,), jnp.float32),
        pltpu.SemaphoreType.DMA,
        pltpu.SemaphoreType.DMA,
        pltpu.SemaphoreType.DMA,
        pltpu.SemaphoreType.DMA,
    ],
    compiler_params=pltpu.CompilerParams(needs_layout_passes=False),
)(_score_body)


def kernel(x, pos_edge_index, neg_edge_index, W, b):
    Wh = _matmul(x, W, b)

    zrow = jnp.zeros((N_NODES, D), jnp.float32)
    zdeg = jnp.zeros((N_NODES,), jnp.float32)
    ones = jnp.ones((CHA,), jnp.float32)

    idx_agg = jnp.stack(
        [
            pos_edge_index[0].reshape(NW, NCH_A, CHA),
            pos_edge_index[1].reshape(NW, NCH_A, CHA),
        ],
        axis=2,
    )
    sum_p, deg_p = _agg(Wh, idx_agg, zrow, zdeg, ones)
    h = _combine(sum_p, deg_p.reshape(NC, N_NODES, 1))

    all_src = jnp.concatenate([pos_edge_index[0], neg_edge_index[0]])
    all_dst = jnp.concatenate([pos_edge_index[1], neg_edge_index[1]])
    scores = _score(h, all_src, all_dst)

    pos_score = scores[:N_EDGES, None]
    neg_score = scores[N_EDGES:, None]
    return (pos_score, neg_score)


# speedup vs baseline: 8.4361x; 1.0939x over previous
import functools

import jax
import jax.numpy as jnp
from jax import lax
from jax.experimental import pallas as pl
from jax.experimental.pallas import tpu as pltpu
from jax.experimental.pallas import tpu_sc as plsc

N_NODES = 10000
N_EDGES = 320000
D = 128

NC = 2
NS = 16
NW = NC * NS

EPT_AGG = N_EDGES // NW
EPT_SCO = (2 * N_EDGES) // NW
CHA = 125
NCH_A = EPT_AGG // CHA
CHS = 200
NCH_S = EPT_SCO // CHS
ZROWS = 1000
NZT = N_NODES // ZROWS

_mesh = plsc.VectorSubcoreMesh(
    core_axis_name="c", subcore_axis_name="s", num_cores=NC, num_subcores=NS
)


def _mm_body(x_ref, w_ref, b_ref, o_ref):
    o_ref[...] = (
        jnp.dot(x_ref[...], w_ref[...], preferred_element_type=jnp.float32)
        + b_ref[...]
    )


def _matmul(x, W, b):
    return pl.pallas_call(
        _mm_body,
        out_shape=jax.ShapeDtypeStruct((N_NODES, D), jnp.float32),
        grid=(10,),
        in_specs=[
            pl.BlockSpec((N_NODES // 10, D), lambda i: (i, 0)),
            pl.BlockSpec((D, D), lambda i: (0, 0)),
            pl.BlockSpec((1, D), lambda i: (0, 0)),
        ],
        out_specs=pl.BlockSpec((N_NODES // 10, D), lambda i: (i, 0)),
    )(x, W, b.reshape(1, D))


def _agg_body(
    wh_hbm, idx_hbm, zrow_hbm, zdeg_hbm, ones_hbm,
    sum_out, deg_out,
    acc_sh, deg_sh,
    idx0, idx1, rows0, rows1, ones_v, deg_v,
    sI0, sI1, sG0, sG1, sS0, sS1, sD0, sD1,
):
    c = lax.axis_index("c")
    s = lax.axis_index("s")
    wid = s * NC + c

    @pl.when(s < NZT)
    def _():
        pltpu.sync_copy(
            zrow_hbm.at[pl.ds(s * ZROWS, ZROWS)],
            acc_sh.at[pl.ds(s * ZROWS, ZROWS)],
        )

    @pl.when(s == NZT)
    def _():
        for k in range(N_NODES // 1000):
            pltpu.sync_copy(zdeg_hbm.at[pl.ds(k * 1000, 1000)], deg_v)
            pltpu.sync_copy(deg_v, deg_sh.at[pl.ds(k * 1000, 1000)])

    pltpu.sync_copy(ones_hbm, ones_v)
    plsc.subcore_barrier()

    def icopy(j, idxb, si):
        pltpu.async_copy(idx_hbm.at[wid, j], idxb, si)

    def iwait(j, idxb, si):
        pltpu.make_async_copy(idx_hbm.at[wid, j], idxb, si).wait()

    def gstart(idxb, rows, sg):
        pltpu.async_copy(wh_hbm.at[idxb.at[0]], rows, sg)

    def gwait(idxb, rows, sg):
        pltpu.make_async_copy(wh_hbm.at[idxb.at[0]], rows, sg).wait()

    def sstart(idxb, rows, ss, sd):
        pltpu.async_copy(rows, acc_sh.at[idxb.at[1]], ss, add=True)
        pltpu.async_copy(ones_v, deg_sh.at[idxb.at[1]], sd, add=True)

    def swait(idxb, rows, ss, sd):
        pltpu.make_async_copy(rows, acc_sh.at[idxb.at[1]], ss).wait()
        pltpu.make_async_copy(ones_v, deg_sh.at[idxb.at[1]], sd).wait()

    icopy(0, idx0, sI0)
    iwait(0, idx0, sI0)
    gstart(idx0, rows0, sG0)

    def body(p, carry):
        i0 = 2 * p
        i1 = i0 + 1
        i2 = i0 + 2

        @pl.when(p > 0)
        def _():
            swait(idx1, rows1, sS1, sD1)

        icopy(i1, idx1, sI1)
        gwait(idx0, rows0, sG0)
        iwait(i1, idx1, sI1)
        gstart(idx1, rows1, sG1)
        sstart(idx0, rows0, sS0, sD0)
        gwait(idx1, rows1, sG1)
        swait(idx0, rows0, sS0, sD0)

        @pl.when(i2 < NCH_A)
        def _():
            icopy(i2, idx0, sI0)
            iwait(i2, idx0, sI0)
            gstart(idx0, rows0, sG0)

        sstart(idx1, rows1, sS1, sD1)
        return carry

    lax.fori_loop(0, NCH_A // 2, body, 0)
    swait(idx1, rows1, sS1, sD1)
    plsc.subcore_barrier()

    @pl.when(s < NZT)
    def _():
        pltpu.sync_copy(
            acc_sh.at[pl.ds(s * ZROWS, ZROWS)],
            sum_out.at[c, pl.ds(s * ZROWS, ZROWS)],
        )

    @pl.when(s == NZT)
    def _():
        for k in range(N_NODES // 1000):
            pltpu.sync_copy(deg_sh.at[pl.ds(k * 1000, 1000)], deg_v)
            pltpu.sync_copy(
                deg_v, deg_out.at[pl.ds(c * N_NODES + k * 1000, 1000)]
            )


_agg = functools.partial(
    pl.kernel,
    out_type=(
        jax.ShapeDtypeStruct((NC, N_NODES, D), jnp.float32),
        jax.ShapeDtypeStruct((NC * N_NODES,), jnp.float32),
    ),
    mesh=_mesh,
    scratch_types=[
        pltpu.VMEM_SHARED((N_NODES, D), jnp.float32),
        pltpu.VMEM_SHARED((N_NODES,), jnp.float32),
        pltpu.VMEM((2, CHA), jnp.int32),
        pltpu.VMEM((2, CHA), jnp.int32),
        pltpu.VMEM((CHA, D), jnp.float32),
        pltpu.VMEM((CHA, D), jnp.float32),
        pltpu.VMEM((CHA,), jnp.float32),
        pltpu.VMEM((1000,), jnp.float32),
        pltpu.SemaphoreType.DMA,
        pltpu.SemaphoreType.DMA,
        pltpu.SemaphoreType.DMA,
        pltpu.SemaphoreType.DMA,
        pltpu.SemaphoreType.DMA,
        pltpu.SemaphoreType.DMA,
        pltpu.SemaphoreType.DMA,
        pltpu.SemaphoreType.DMA,
    ],
)(_agg_body)


def _comb_body(s_ref, d_ref, o_ref):
    deg = d_ref[0] + d_ref[1]
    o_ref[...] = (s_ref[0] + s_ref[1]) / jnp.maximum(deg, 1.0)


def _combine(sum_p, deg_p):
    blk = N_NODES // 10
    return pl.pallas_call(
        _comb_body,
        out_shape=jax.ShapeDtypeStruct((N_NODES, D), jnp.float32),
        grid=(10,),
        in_specs=[
            pl.BlockSpec((NC, blk, D), lambda i: (0, i, 0)),
            pl.BlockSpec((NC, blk, 1), lambda i: (0, i, 0)),
        ],
        out_specs=pl.BlockSpec((blk, D), lambda i: (i, 0)),
    )(sum_p, deg_p)


NFG = CHS // 16
NTAIL = CHS - NFG * 16


def _score_body(
    h_hbm, idx_hbm,
    out_hbm,
    idx0, idx1, ra0, rb0, ra1, rb1, accs, sc_buf,
    sI0, sI1, sA0, sB0, sA1, sB1,
):
    c = lax.axis_index("c")
    s = lax.axis_index("s")
    wid = s * NC + c
    base = wid * EPT_SCO
    lane = lax.broadcasted_iota(jnp.int32, (16,), 0)
    col = lane * 16

    def icopy(j, idxb, si):
        pltpu.async_copy(idx_hbm.at[wid, j], idxb, si)

    def iwait(j, idxb, si):
        pltpu.make_async_copy(idx_hbm.at[wid, j], idxb, si).wait()

    H = CHS // 2

    def gstart(idxb, ra, rb, sa, sb):
        pltpu.async_copy(h_hbm.at[idxb.at[0]], ra.at[pl.ds(0, H)], sa)
        pltpu.async_copy(h_hbm.at[idxb.at[1]], ra.at[pl.ds(H, H)], sa)
        pltpu.async_copy(h_hbm.at[idxb.at[2]], rb.at[pl.ds(0, H)], sb)
        pltpu.async_copy(h_hbm.at[idxb.at[3]], rb.at[pl.ds(H, H)], sb)

    def gwait(idxb, ra, rb, sa, sb):
        pltpu.make_async_copy(h_hbm.at[idxb.at[0]], ra, sa).wait()
        pltpu.make_async_copy(h_hbm.at[idxb.at[2]], rb, sb).wait()

    def dot16(ra, rb, g, n_edges):
        for e in range(n_edges):
            r = g * 16 + e
            acc = ra[r, pl.ds(0, 16)] * rb[r, pl.ds(0, 16)]
            for k in range(1, 8):
                acc = acc + ra[r, pl.ds(k * 16, 16)] * rb[r, pl.ds(k * 16, 16)]
            accs[pl.ds(e * 16, 16)] = acc
        tot = plsc.load_gather(accs, [col])
        for jj in range(1, 16):
            tot = tot + plsc.load_gather(accs, [col + jj])
        return tot

    def compute(j, ra, rb):
        def grp(g, carry):
            tot = dot16(ra, rb, g, 16)
            sc_buf[pl.ds(j * CHS + g * 16, 16)] = tot
            return carry

        lax.fori_loop(0, NFG, grp, 0)
        if NTAIL:
            tot = dot16(ra, rb, NFG, NTAIL)
            plsc.store_scatter(
                sc_buf, [j * CHS + NFG * 16 + lane], tot, mask=lane < NTAIL
            )

    icopy(0, idx0, sI0)
    iwait(0, idx0, sI0)
    gstart(idx0, ra0, rb0, sA0, sB0)
    icopy(1, idx1, sI1)

    def body(p, carry):
        i0 = 2 * p
        i1 = i0 + 1
        i2 = i0 + 2
        i3 = i0 + 3

        iwait(i1, idx1, sI1)
        gstart(idx1, ra1, rb1, sA1, sB1)
        gwait(idx0, ra0, rb0, sA0, sB0)

        @pl.when(i2 < NCH_S)
        def _():
            icopy(i2, idx0, sI0)

        compute(i0, ra0, rb0)

        @pl.when(i2 < NCH_S)
        def _():
            iwait(i2, idx0, sI0)
            gstart(idx0, ra0, rb0, sA0, sB0)

        gwait(idx1, ra1, rb1, sA1, sB1)

        @pl.when(i3 < NCH_S)
        def _():
            icopy(i3, idx1, sI1)

        compute(i1, ra1, rb1)
        return carry

    lax.fori_loop(0, NCH_S // 2, body, 0)
    pltpu.sync_copy(sc_buf, out_hbm.at[pl.ds(base, EPT_SCO)])


_score = functools.partial(
    pl.kernel,
    out_type=jax.ShapeDtypeStruct((2 * N_EDGES,), jnp.float32),
    mesh=_mesh,
    scratch_types=[
        pltpu.VMEM((4, CHS // 2), jnp.int32),
        pltpu.VMEM((4, CHS // 2), jnp.int32),
        pltpu.VMEM((CHS, D), jnp.float32),
        pltpu.VMEM((CHS, D), jnp.float32),
        pltpu.VMEM((CHS, D), jnp.float32),
        pltpu.VMEM((CHS, D), jnp.float32),
        pltpu.VMEM((16 * 16,), jnp.float32),
        pltpu.VMEM((EPT_SCO,), jnp.float32),
        pltpu.SemaphoreType.DMA,
        pltpu.SemaphoreType.DMA,
        pltpu.SemaphoreType.DMA,
        pltpu.SemaphoreType.DMA,
        pltpu.SemaphoreType.DMA,
        pltpu.SemaphoreType.DMA,
    ],
    compiler_params=pltpu.CompilerParams(needs_layout_passes=False),
)(_score_body)


def kernel(x, pos_edge_index, neg_edge_index, W, b):
    Wh = _matmul(x, W, b)

    zrow = jnp.zeros((N_NODES, D), jnp.float32)
    zdeg = jnp.zeros((N_NODES,), jnp.float32)
    ones = jnp.ones((CHA,), jnp.float32)

    idx_agg = jnp.stack(
        [
            pos_edge_index[0].reshape(NW, NCH_A, CHA),
            pos_edge_index[1].reshape(NW, NCH_A, CHA),
        ],
        axis=2,
    )
    sum_p, deg_p = _agg(Wh, idx_agg, zrow, zdeg, ones)
    h = _combine(sum_p, deg_p.reshape(NC, N_NODES, 1))

    all_src = jnp.concatenate([pos_edge_index[0], neg_edge_index[0]])
    all_dst = jnp.concatenate([pos_edge_index[1], neg_edge_index[1]])
    idx_sco = jnp.concatenate(
        [
            all_src.reshape(NW, NCH_S, 2, CHS // 2),
            all_dst.reshape(NW, NCH_S, 2, CHS // 2),
        ],
        axis=2,
    )
    scores = _score(h, idx_sco)

    pos_score = scores[:N_EDGES, None]
    neg_score = scores[N_EDGES:, None]
    return (pos_score, neg_score)
